# baseline (speedup 1.0000x reference)
import functools

import jax
import jax.numpy as jnp
from jax import lax
from jax.experimental import pallas as pl
from jax.experimental.pallas import tpu as pltpu
from jax.experimental.pallas import tpu_sc as plsc

_C = 16
_N = 1048576
_NW = 32
_COLS = _N // _NW
_CH = 16384
_K = _COLS // _CH


def _sc_partials(lossTensor, lcSuccesses):
    mesh = plsc.VectorSubcoreMesh(core_axis_name="c", subcore_axis_name="s")

    @functools.partial(
        pl.kernel,
        mesh=mesh,
        out_type=jax.ShapeDtypeStruct((_NW, 2 * _C, 16), jnp.float32),
        scratch_types=[
            pltpu.VMEM((2, _CH), jnp.float32),
            pltpu.VMEM((2, _CH), jnp.int32),
            pltpu.VMEM((2 * _C, 16), jnp.float32),
            pltpu.SemaphoreType.DMA,
            pltpu.SemaphoreType.DMA,
        ],
    )
    def k(loss_hbm, succ_hbm, out_hbm, xb, sb, res_v, sem0, sem1):
        wid = lax.axis_index("s") * 2 + lax.axis_index("c")
        base = wid * _COLS
        sems = (sem0, sem1)

        steps = []
        for r in range(_C):
            for ch in range(_K):
                steps.append((r, ch))
        tot = len(steps)

        def start(it):
            r, ch = steps[it]
            slot = it % 2
            src_x = loss_hbm.at[r, pl.ds(base + ch * _CH, _CH)]
            src_s = succ_hbm.at[r, pl.ds(base + ch * _CH, _CH)]
            hx = pltpu.async_copy(src_x, xb.at[slot], sems[slot])
            hs = pltpu.async_copy(src_s, sb.at[slot], sems[slot])
            return (hx, hs)

        def compute(it, handles):
            r, ch = steps[it]
            slot = it % 2
            hx, hs = handles
            hx.wait()
            hs.wait()

            def body(i, carry):
                at, aa = carry
                v = xb[slot, pl.ds(i * 16, 16)]
                s = sb[slot, pl.ds(i * 16, 16)]
                at = at + jnp.where(s == 1, v, 0.0)
                aa = aa + v
                return (at, aa)

            z = jnp.zeros((16,), jnp.float32)
            at, aa = lax.fori_loop(0, _CH // 16, body, (z, z), unroll=8)
            if ch == 0:
                res_v[r, :] = at
                res_v[_C + r, :] = aa
            else:
                res_v[r, :] = res_v[r, :] + at
                res_v[_C + r, :] = res_v[_C + r, :] + aa

        pending = start(0)
        for it in range(tot):
            nxt = start(it + 1) if it + 1 < tot else None
            compute(it, pending)
            pending = nxt

        pltpu.sync_copy(res_v, out_hbm.at[wid])

    return k(lossTensor, lcSuccesses)


def _stage2(pt_ref, pa_ref, out_ref):
    ts = jnp.sum(pt_ref[...], axis=1, keepdims=True)
    tt = jnp.sum(pa_ref[...], axis=1, keepdims=True)
    lv = jnp.log(ts / tt)
    kl = jnp.maximum(lv * lv - 0.01, 0.0)
    out_ref[...] = jnp.sum(kl, axis=0, keepdims=True)


def _finish(partials):
    pt = partials[:, 0:_C, :].transpose(1, 0, 2).reshape(_C, _NW * 16)
    pa = partials[:, _C:2 * _C, :].transpose(1, 0, 2).reshape(_C, _NW * 16)
    out = pl.pallas_call(
        _stage2,
        in_specs=[
            pl.BlockSpec((_C, _NW * 16), lambda: (0, 0)),
            pl.BlockSpec((_C, _NW * 16), lambda: (0, 0)),
        ],
        out_specs=pl.BlockSpec((1, 1), lambda: (0, 0)),
        out_shape=jax.ShapeDtypeStruct((1, 1), jnp.float32),
    )(pt, pa)
    return out[0, 0]


def kernel(lossTensor, lcSuccesses):
    return _finish(_sc_partials(lossTensor, lcSuccesses))


# speedup vs baseline: 1.7575x; 1.7575x over previous
import functools

import jax
import jax.numpy as jnp
from jax import lax
from jax.experimental import pallas as pl
from jax.experimental.pallas import tpu as pltpu
from jax.experimental.pallas import tpu_sc as plsc

_C = 16
_N = 1048576
_NW = 32
_SC_COLS = 8192
_CH = 8192
_K = _SC_COLS // _CH
_NT = _N - _NW * _SC_COLS
_BLK = 65536


def _sc_partials(lossTensor, lcSuccesses):
    mesh = plsc.VectorSubcoreMesh(core_axis_name="c", subcore_axis_name="s")

    @functools.partial(
        pl.kernel,
        mesh=mesh,
        out_type=jax.ShapeDtypeStruct((_NW, 2 * _C, 16), jnp.float32),
        scratch_types=[
            pltpu.VMEM((2, _CH), jnp.float32),
            pltpu.VMEM((2, _CH), jnp.int32),
            pltpu.VMEM((2 * _C, 16), jnp.float32),
            pltpu.SemaphoreType.DMA,
            pltpu.SemaphoreType.DMA,
        ],
    )
    def k(loss_hbm, succ_hbm, out_hbm, xb, sb, res_v, sem0, sem1):
        wid = lax.axis_index("s") * 2 + lax.axis_index("c")
        base = _NT + wid * _SC_COLS
        sems = (sem0, sem1)

        steps = []
        for r in range(_C):
            for ch in range(_K):
                steps.append((r, ch))
        tot = len(steps)

        def start(it):
            r, ch = steps[it]
            slot = it % 2
            src_x = loss_hbm.at[r, pl.ds(base + ch * _CH, _CH)]
            src_s = succ_hbm.at[r, pl.ds(base + ch * _CH, _CH)]
            hx = pltpu.async_copy(src_x, xb.at[slot], sems[slot])
            hs = pltpu.async_copy(src_s, sb.at[slot], sems[slot])
            return (hx, hs)

        def compute(it, handles):
            r, ch = steps[it]
            slot = it % 2
            hx, hs = handles
            hx.wait()
            hs.wait()

            def body(i, carry):
                at, aa = carry
                v = xb[slot, pl.ds(i * 16, 16)]
                s = sb[slot, pl.ds(i * 16, 16)]
                at = at + jnp.where(s == 1, v, 0.0)
                aa = aa + v
                return (at, aa)

            z = jnp.zeros((16,), jnp.float32)
            at, aa = lax.fori_loop(0, _CH // 16, body, (z, z), unroll=8)
            if ch == 0:
                res_v[r, :] = at
                res_v[_C + r, :] = aa
            else:
                res_v[r, :] = res_v[r, :] + at
                res_v[_C + r, :] = res_v[_C + r, :] + aa

        pending = start(0)
        for it in range(tot):
            nxt = start(it + 1) if it + 1 < tot else None
            compute(it, pending)
            pending = nxt

        pltpu.sync_copy(res_v, out_hbm.at[wid])

    return k(lossTensor, lcSuccesses)


def _tc_fold(x):
    acc = x[:, 0:128]
    for l in range(1, _BLK // 128):
        acc = acc + x[:, 128 * l:128 * (l + 1)]
    return acc


def _tc_body(loss_ref, succ_ref, pt_ref, pa_ref, at_ref, aa_ref):
    i = pl.program_id(0)

    @pl.when(i == 0)
    def _init():
        at_ref[...] = jnp.zeros_like(at_ref)
        aa_ref[...] = jnp.zeros_like(aa_ref)

    x = loss_ref[...]
    masked = jnp.where(succ_ref[...] == 1, x, 0.0)
    at_ref[...] += _tc_fold(masked)
    aa_ref[...] += _tc_fold(x)

    @pl.when(i == pl.num_programs(0) - 1)
    def _fini():
        pt_ref[...] = at_ref[...]
        pa_ref[...] = aa_ref[...]


def _tc_partials(lossTensor, lcSuccesses):
    grid = _NT // _BLK
    return pl.pallas_call(
        _tc_body,
        grid=(grid,),
        in_specs=[
            pl.BlockSpec((_C, _BLK), lambda i: (0, i)),
            pl.BlockSpec((_C, _BLK), lambda i: (0, i)),
        ],
        out_specs=[
            pl.BlockSpec((_C, 128), lambda i: (0, 0)),
            pl.BlockSpec((_C, 128), lambda i: (0, 0)),
        ],
        out_shape=[
            jax.ShapeDtypeStruct((_C, 128), jnp.float32),
            jax.ShapeDtypeStruct((_C, 128), jnp.float32),
        ],
        scratch_shapes=[
            pltpu.VMEM((_C, 128), jnp.float32),
            pltpu.VMEM((_C, 128), jnp.float32),
        ],
        compiler_params=pltpu.CompilerParams(
            dimension_semantics=("arbitrary",),
        ),
    )(lossTensor, lcSuccesses)


def _stage2(pt_tc_ref, pa_tc_ref, pt_sc_ref, pa_sc_ref, out_ref):
    ts = (jnp.sum(pt_tc_ref[...], axis=1, keepdims=True)
          + jnp.sum(pt_sc_ref[...], axis=1, keepdims=True))
    tt = (jnp.sum(pa_tc_ref[...], axis=1, keepdims=True)
          + jnp.sum(pa_sc_ref[...], axis=1, keepdims=True))
    lv = jnp.log(ts / tt)
    kl = jnp.maximum(lv * lv - 0.01, 0.0)
    out_ref[...] = jnp.sum(kl, axis=0, keepdims=True)


def kernel(lossTensor, lcSuccesses):
    sc = _sc_partials(lossTensor, lcSuccesses)
    pt_tc, pa_tc = _tc_partials(lossTensor, lcSuccesses)
    pt_sc = sc[:, 0:_C, :].transpose(1, 0, 2).reshape(_C, _NW * 16)
    pa_sc = sc[:, _C:2 * _C, :].transpose(1, 0, 2).reshape(_C, _NW * 16)
    out = pl.pallas_call(
        _stage2,
        in_specs=[
            pl.BlockSpec((_C, 128), lambda: (0, 0)),
            pl.BlockSpec((_C, 128), lambda: (0, 0)),
            pl.BlockSpec((_C, _NW * 16), lambda: (0, 0)),
            pl.BlockSpec((_C, _NW * 16), lambda: (0, 0)),
        ],
        out_specs=pl.BlockSpec((1, 1), lambda: (0, 0)),
        out_shape=jax.ShapeDtypeStruct((1, 1), jnp.float32),
    )(pt_tc, pa_tc, pt_sc, pa_sc)
    return out[0, 0]


# speedup vs baseline: 1.7612x; 1.0021x over previous
import functools

import jax
import jax.numpy as jnp
from jax import lax
from jax.experimental import pallas as pl
from jax.experimental.pallas import tpu as pltpu
from jax.experimental.pallas import tpu_sc as plsc

_C = 16
_N = 1048576
_NW = 32
_SC_COLS = 2048
_CH = 2048
_K = _SC_COLS // _CH
_NT = _N - _NW * _SC_COLS
_BLK = 65536


def _sc_partials(lossTensor, lcSuccesses):
    mesh = plsc.VectorSubcoreMesh(core_axis_name="c", subcore_axis_name="s")

    @functools.partial(
        pl.kernel,
        mesh=mesh,
        out_type=jax.ShapeDtypeStruct((_NW, 2 * _C, 16), jnp.float32),
        scratch_types=[
            pltpu.VMEM((2, _CH), jnp.float32),
            pltpu.VMEM((2, _CH), jnp.int32),
            pltpu.VMEM((2 * _C, 16), jnp.float32),
            pltpu.SemaphoreType.DMA,
            pltpu.SemaphoreType.DMA,
        ],
    )
    def k(loss_hbm, succ_hbm, out_hbm, xb, sb, res_v, sem0, sem1):
        wid = lax.axis_index("s") * 2 + lax.axis_index("c")
        base = _NT + wid * _SC_COLS
        sems = (sem0, sem1)

        steps = []
        for r in range(_C):
            for ch in range(_K):
                steps.append((r, ch))
        tot = len(steps)

        def start(it):
            r, ch = steps[it]
            slot = it % 2
            src_x = loss_hbm.at[r, pl.ds(base + ch * _CH, _CH)]
            src_s = succ_hbm.at[r, pl.ds(base + ch * _CH, _CH)]
            hx = pltpu.async_copy(src_x, xb.at[slot], sems[slot])
            hs = pltpu.async_copy(src_s, sb.at[slot], sems[slot])
            return (hx, hs)

        def compute(it, handles):
            r, ch = steps[it]
            slot = it % 2
            hx, hs = handles
            hx.wait()
            hs.wait()

            def body(i, carry):
                at, aa = carry
                v = xb[slot, pl.ds(i * 16, 16)]
                s = sb[slot, pl.ds(i * 16, 16)]
                at = at + jnp.where(s == 1, v, 0.0)
                aa = aa + v
                return (at, aa)

            z = jnp.zeros((16,), jnp.float32)
            at, aa = lax.fori_loop(0, _CH // 16, body, (z, z), unroll=8)
            if ch == 0:
                res_v[r, :] = at
                res_v[_C + r, :] = aa
            else:
                res_v[r, :] = res_v[r, :] + at
                res_v[_C + r, :] = res_v[_C + r, :] + aa

        pending = start(0)
        for it in range(tot):
            nxt = start(it + 1) if it + 1 < tot else None
            compute(it, pending)
            pending = nxt

        pltpu.sync_copy(res_v, out_hbm.at[wid])

    return k(lossTensor, lcSuccesses)


def _tc_fold(x):
    acc = x[:, 0:128]
    for l in range(1, _BLK // 128):
        acc = acc + x[:, 128 * l:128 * (l + 1)]
    return acc


def _tc_body(loss_ref, succ_ref, pt_ref, pa_ref, at_ref, aa_ref):
    i = pl.program_id(0)

    @pl.when(i == 0)
    def _init():
        at_ref[...] = jnp.zeros_like(at_ref)
        aa_ref[...] = jnp.zeros_like(aa_ref)

    x = loss_ref[...]
    masked = jnp.where(succ_ref[...] == 1, x, 0.0)
    at_ref[...] += _tc_fold(masked)
    aa_ref[...] += _tc_fold(x)

    @pl.when(i == pl.num_programs(0) - 1)
    def _fini():
        pt_ref[...] = at_ref[...]
        pa_ref[...] = aa_ref[...]


def _tc_partials(lossTensor, lcSuccesses):
    grid = _NT // _BLK
    return pl.pallas_call(
        _tc_body,
        grid=(grid,),
        in_specs=[
            pl.BlockSpec((_C, _BLK), lambda i: (0, i)),
            pl.BlockSpec((_C, _BLK), lambda i: (0, i)),
        ],
        out_specs=[
            pl.BlockSpec((_C, 128), lambda i: (0, 0)),
            pl.BlockSpec((_C, 128), lambda i: (0, 0)),
        ],
        out_shape=[
            jax.ShapeDtypeStruct((_C, 128), jnp.float32),
            jax.ShapeDtypeStruct((_C, 128), jnp.float32),
        ],
        scratch_shapes=[
            pltpu.VMEM((_C, 128), jnp.float32),
            pltpu.VMEM((_C, 128), jnp.float32),
        ],
        compiler_params=pltpu.CompilerParams(
            dimension_semantics=("arbitrary",),
        ),
    )(lossTensor, lcSuccesses)


def _stage2(pt_tc_ref, pa_tc_ref, pt_sc_ref, pa_sc_ref, out_ref):
    ts = (jnp.sum(pt_tc_ref[...], axis=1, keepdims=True)
          + jnp.sum(pt_sc_ref[...], axis=1, keepdims=True))
    tt = (jnp.sum(pa_tc_ref[...], axis=1, keepdims=True)
          + jnp.sum(pa_sc_ref[...], axis=1, keepdims=True))
    lv = jnp.log(ts / tt)
    kl = jnp.maximum(lv * lv - 0.01, 0.0)
    out_ref[...] = jnp.sum(kl, axis=0, keepdims=True)


def kernel(lossTensor, lcSuccesses):
    sc = _sc_partials(lossTensor, lcSuccesses)
    pt_tc, pa_tc = _tc_partials(lossTensor, lcSuccesses)
    pt_sc = sc[:, 0:_C, :].transpose(1, 0, 2).reshape(_C, _NW * 16)
    pa_sc = sc[:, _C:2 * _C, :].transpose(1, 0, 2).reshape(_C, _NW * 16)
    out = pl.pallas_call(
        _stage2,
        in_specs=[
            pl.BlockSpec((_C, 128), lambda: (0, 0)),
            pl.BlockSpec((_C, 128), lambda: (0, 0)),
            pl.BlockSpec((_C, _NW * 16), lambda: (0, 0)),
            pl.BlockSpec((_C, _NW * 16), lambda: (0, 0)),
        ],
        out_specs=pl.BlockSpec((1, 1), lambda: (0, 0)),
        out_shape=jax.ShapeDtypeStruct((1, 1), jnp.float32),
    )(pt_tc, pa_tc, pt_sc, pa_sc)
    return out[0, 0]


# speedup vs baseline: 2.6221x; 1.4888x over previous
import jax
import jax.numpy as jnp
from jax.experimental import pallas as pl
from jax.experimental.pallas import tpu as pltpu

_C = 16
_N = 1048576
_BLK = 65536


def _fold(x):
    acc = x[:, 0:128]
    for l in range(1, _BLK // 128):
        acc = acc + x[:, 128 * l:128 * (l + 1)]
    return acc


def _body(loss_ref, succ_ref, out_ref, at_ref, aa_ref):
    i = pl.program_id(0)

    @pl.when(i == 0)
    def _init():
        at_ref[...] = jnp.zeros_like(at_ref)
        aa_ref[...] = jnp.zeros_like(aa_ref)

    x = loss_ref[...]
    masked = jnp.where(succ_ref[...] == 1, x, 0.0)
    at_ref[...] += _fold(masked)
    aa_ref[...] += _fold(x)

    @pl.when(i == pl.num_programs(0) - 1)
    def _fini():
        ts = jnp.sum(at_ref[...], axis=1, keepdims=True)
        tt = jnp.sum(aa_ref[...], axis=1, keepdims=True)
        lv = jnp.log(ts / tt)
        kl = jnp.maximum(lv * lv - 0.01, 0.0)
        out_ref[...] = jnp.sum(kl, axis=0, keepdims=True)


def kernel(lossTensor, lcSuccesses):
    grid = _N // _BLK
    out = pl.pallas_call(
        _body,
        grid=(grid,),
        in_specs=[
            pl.BlockSpec((_C, _BLK), lambda i: (0, i)),
            pl.BlockSpec((_C, _BLK), lambda i: (0, i)),
        ],
        out_specs=pl.BlockSpec((1, 1), lambda i: (0, 0)),
        out_shape=jax.ShapeDtypeStruct((1, 1), jnp.float32),
        scratch_shapes=[
            pltpu.VMEM((_C, 128), jnp.float32),
            pltpu.VMEM((_C, 128), jnp.float32),
        ],
        compiler_params=pltpu.CompilerParams(
            dimension_semantics=("arbitrary",),
        ),
    )(lossTensor, lcSuccesses)
    return out[0, 0]
